# Initial kernel scaffold; baseline (speedup 1.0000x reference)
import functools

import jax
import jax.numpy as jnp
from jax import lax
from jax.experimental import pallas as pl
from jax.experimental.pallas import tpu as pltpu
from jax.experimental.pallas import tpu_sc as plsc

ALPHA = 0.1
K_STEPS = 10

NC = 2
NS = 16
NW = NC * NS
CH = 128


def _ceil_div(a, b):
    return -(-a // b)


def _pick_block(n):
    for b in (2000, 2500, 1250, 1000, 500, 200, 100, 50, 25, 10, 8):
        if n % b == 0:
            return b
    return n


def _mlp_body(x_ref, w1_ref, b1_ref, w2_ref, b2_ref, h_ref):
    h1 = jnp.maximum(
        jnp.dot(x_ref[...], w1_ref[...], preferred_element_type=jnp.float32)
        + b1_ref[...], 0.0)
    h_ref[...] = (
        jnp.dot(h1, w2_ref[...], preferred_element_type=jnp.float32)
        + b2_ref[...])


def _mlp(x, W1, b1, W2, b2):
    n, d_in = x.shape
    d_h = W1.shape[1]
    d_out = W2.shape[1]
    bn = _pick_block(n)
    return pl.pallas_call(
        _mlp_body,
        grid=(n // bn,),
        in_specs=[
            pl.BlockSpec((bn, d_in), lambda i: (i, 0)),
            pl.BlockSpec((d_in, d_h), lambda i: (0, 0)),
            pl.BlockSpec((1, d_h), lambda i: (0, 0)),
            pl.BlockSpec((d_h, d_out), lambda i: (0, 0)),
            pl.BlockSpec((1, d_out), lambda i: (0, 0)),
        ],
        out_specs=pl.BlockSpec((bn, d_out), lambda i: (i, 0)),
        out_shape=jax.ShapeDtypeStruct((n, d_out), jnp.float32),
    )(x, W1, b1.reshape(1, -1), W2, b2.reshape(1, -1))


def _dinv_g0_body(degp_ref, h_ref, dinv_ref, g_ref):
    deg = degp_ref[0] + degp_ref[1] + 1.0
    dinv = lax.rsqrt(deg)
    dinv_ref[...] = dinv
    g_ref[...] = dinv * h_ref[...]


def _dinv_g0(degp, h):
    n, d = h.shape
    bn = _pick_block(n)
    return pl.pallas_call(
        _dinv_g0_body,
        grid=(n // bn,),
        in_specs=[
            pl.BlockSpec((2, bn, d), lambda i: (0, i, 0)),
            pl.BlockSpec((bn, d), lambda i: (i, 0)),
        ],
        out_specs=[
            pl.BlockSpec((bn, d), lambda i: (i, 0)),
            pl.BlockSpec((bn, d), lambda i: (i, 0)),
        ],
        out_shape=[
            jax.ShapeDtypeStruct((n, d), jnp.float32),
            jax.ShapeDtypeStruct((n, d), jnp.float32),
        ],
    )(degp, h)


def _combine_body(sp_ref, g_ref, dinv_ref, x0_ref, h_ref, gn_ref):
    s = sp_ref[0] + sp_ref[1] + g_ref[...]
    h = (1.0 - ALPHA) * (dinv_ref[...] * s) + ALPHA * x0_ref[...]
    h_ref[...] = h
    gn_ref[...] = dinv_ref[...] * h


def _combine(sp, g, dinv, x0):
    n, d = g.shape
    bn = _pick_block(n)
    return pl.pallas_call(
        _combine_body,
        grid=(n // bn,),
        in_specs=[
            pl.BlockSpec((2, bn, d), lambda i: (0, i, 0)),
            pl.BlockSpec((bn, d), lambda i: (i, 0)),
            pl.BlockSpec((bn, d), lambda i: (i, 0)),
            pl.BlockSpec((bn, d), lambda i: (i, 0)),
        ],
        out_specs=[
            pl.BlockSpec((bn, d), lambda i: (i, 0)),
            pl.BlockSpec((bn, d), lambda i: (i, 0)),
        ],
        out_shape=[
            jax.ShapeDtypeStruct((n, d), jnp.float32),
            jax.ShapeDtypeStruct((n, d), jnp.float32),
        ],
    )(sp, g, dinv, x0)


def _zero_shared(zbuf, shared, sid, rows_per_tile):
    @pl.loop(0, 128)
    def _(i):
        zbuf[i, :] = jnp.zeros((16,), jnp.float32)
    base = sid * rows_per_tile
    full = rows_per_tile // 128
    rem = rows_per_tile - full * 128

    @pl.loop(0, full)
    def _(j):
        pltpu.sync_copy(zbuf, shared.at[pl.ds(base + j * 128, 128)])
    if rem:
        pltpu.sync_copy(zbuf.at[pl.ds(0, rem)],
                        shared.at[pl.ds(base + full * 128, rem)])


def _writeout(shared, out_hbm, cid, sid, rows_per_tile):
    base = sid * rows_per_tile
    pltpu.sync_copy(shared.at[pl.ds(base, rows_per_tile)],
                    out_hbm.at[cid, pl.ds(base, rows_per_tile)])


def _make_deg_kernel(n_pad, per_w, d):
    rows_per_tile = n_pad // NS
    mesh = plsc.VectorSubcoreMesh(core_axis_name="c", subcore_axis_name="s")

    @functools.partial(
        pl.kernel,
        mesh=mesh,
        out_type=jax.ShapeDtypeStruct((NC, n_pad, d), jnp.float32),
        scratch_types=[
            pltpu.VMEM((per_w, CH), jnp.int32),
            pltpu.VMEM((CH, d), jnp.float32),
            pltpu.VMEM((128, d), jnp.float32),
            pltpu.VMEM_SHARED((n_pad, d), jnp.float32),
        ],
    )
    def deg_kernel(col_hbm, out_hbm, cidx, ones, zbuf, shared):
        cid = lax.axis_index("c")
        sid = lax.axis_index("s")
        w = sid * NC + cid

        _zero_shared(zbuf, shared, sid, rows_per_tile)

        @pl.loop(0, CH)
        def _(i):
            ones[i, :] = jnp.ones((16,), jnp.float32)

        pltpu.sync_copy(col_hbm.at[w], cidx)
        plsc.subcore_barrier()

        @pl.loop(0, per_w)
        def _(j):
            pltpu.sync_copy(ones, shared.at[cidx.at[j]], add=True)

        plsc.subcore_barrier()
        _writeout(shared, out_hbm, cid, sid, rows_per_tile)

    return deg_kernel


def _make_scatter_kernel(n_pad, per_w, d):
    rows_per_tile = n_pad // NS
    mesh = plsc.VectorSubcoreMesh(core_axis_name="c", subcore_axis_name="s")

    @functools.partial(
        pl.kernel,
        mesh=mesh,
        out_type=jax.ShapeDtypeStruct((NC, n_pad, d), jnp.float32),
        scratch_types=[
            pltpu.VMEM((per_w, CH), jnp.int32),
            pltpu.VMEM((per_w, CH), jnp.int32),
            pltpu.VMEM((CH, d), jnp.float32),
            pltpu.VMEM((128, d), jnp.float32),
            pltpu.VMEM_SHARED((n_pad, d), jnp.float32),
            pltpu.SemaphoreType.DMA,
        ],
    )
    def scatter_kernel(g_hbm, row_hbm, col_hbm, out_hbm,
                       ridx, cidx, rows, zbuf, shared, sem):
        cid = lax.axis_index("c")
        sid = lax.axis_index("s")
        w = sid * NC + cid

        _zero_shared(zbuf, shared, sid, rows_per_tile)

        pltpu.sync_copy(row_hbm.at[w], ridx)
        pltpu.sync_copy(col_hbm.at[w], cidx)
        plsc.subcore_barrier()

        @pl.loop(0, per_w)
        def _(j):
            pltpu.async_copy(g_hbm.at[ridx.at[j]], rows, sem).wait()
            pltpu.sync_copy(rows, shared.at[cidx.at[j]], add=True)

        plsc.subcore_barrier()
        _writeout(shared, out_hbm, cid, sid, rows_per_tile)

    return scatter_kernel


def kernel(x, edge_index, W1, b1, W2, b2):
    n = x.shape[0]
    e = edge_index.shape[1]
    d = W2.shape[1]

    row = edge_index[0].astype(jnp.int32)
    col = edge_index[1].astype(jnp.int32)

    per_w = _ceil_div(e, NW * CH)
    e_pad = NW * per_w * CH
    n_pad = _ceil_div(n + 1, NS * 8) * NS * 8

    row_p = jnp.concatenate(
        [row, jnp.zeros((e_pad - e,), jnp.int32)]).reshape(NW, per_w, CH)
    col_p = jnp.concatenate(
        [col, jnp.full((e_pad - e,), n, jnp.int32)]).reshape(NW, per_w, CH)

    deg_k = _make_deg_kernel(n_pad, per_w, d)
    scat_k = _make_scatter_kernel(n_pad, per_w, d)

    h = _mlp(x, W1, b1, W2, b2)
    degp = deg_k(col_p)
    dinv, g = _dinv_g0(degp[:, :n], h)
    x0 = h

    for _ in range(K_STEPS):
        sp = scat_k(g, row_p, col_p)
        h, g = _combine(sp[:, :n], g, dinv, x0)
    return h


# speedup vs baseline: 18.5499x; 18.5499x over previous
import functools

import jax
import jax.numpy as jnp
from jax import lax
from jax.experimental import pallas as pl
from jax.experimental.pallas import tpu as pltpu
from jax.experimental.pallas import tpu_sc as plsc

ALPHA = 0.1
K_STEPS = 10

NC = 2
NS = 16
NW = NC * NS
CH = 128


def _ceil_div(a, b):
    return -(-a // b)


def _pick_block(n):
    for b in (2000, 2500, 1250, 1000, 500, 200, 100, 50, 25, 10, 8):
        if n % b == 0:
            return b
    return n


def _mlp_body(x_ref, w1_ref, b1_ref, w2_ref, b2_ref, h_ref):
    h1 = jnp.maximum(
        jnp.dot(x_ref[...], w1_ref[...], preferred_element_type=jnp.float32)
        + b1_ref[...], 0.0)
    h_ref[...] = (
        jnp.dot(h1, w2_ref[...], preferred_element_type=jnp.float32)
        + b2_ref[...])


def _mlp(x, W1, b1, W2, b2):
    n, d_in = x.shape
    d_h = W1.shape[1]
    d_out = W2.shape[1]
    bn = _pick_block(n)
    return pl.pallas_call(
        _mlp_body,
        grid=(n // bn,),
        in_specs=[
            pl.BlockSpec((bn, d_in), lambda i: (i, 0)),
            pl.BlockSpec((d_in, d_h), lambda i: (0, 0)),
            pl.BlockSpec((1, d_h), lambda i: (0, 0)),
            pl.BlockSpec((d_h, d_out), lambda i: (0, 0)),
            pl.BlockSpec((1, d_out), lambda i: (0, 0)),
        ],
        out_specs=pl.BlockSpec((bn, d_out), lambda i: (i, 0)),
        out_shape=jax.ShapeDtypeStruct((n, d_out), jnp.float32),
    )(x, W1, b1.reshape(1, -1), W2, b2.reshape(1, -1))


def _dinv_g0_body(degp_ref, h_ref, dinv_ref, g_ref):
    deg = degp_ref[0] + degp_ref[1] + 1.0
    dinv = lax.rsqrt(deg)
    dinv_ref[...] = dinv
    g_ref[...] = dinv * h_ref[...]


def _dinv_g0(degp, h):
    n, d = h.shape
    bn = _pick_block(n)
    return pl.pallas_call(
        _dinv_g0_body,
        grid=(n // bn,),
        in_specs=[
            pl.BlockSpec((2, bn, d), lambda i: (0, i, 0)),
            pl.BlockSpec((bn, d), lambda i: (i, 0)),
        ],
        out_specs=[
            pl.BlockSpec((bn, d), lambda i: (i, 0)),
            pl.BlockSpec((bn, d), lambda i: (i, 0)),
        ],
        out_shape=[
            jax.ShapeDtypeStruct((n, d), jnp.float32),
            jax.ShapeDtypeStruct((n, d), jnp.float32),
        ],
    )(degp, h)


def _combine_body(sp_ref, g_ref, dinv_ref, x0_ref, h_ref, gn_ref):
    s = sp_ref[0] + sp_ref[1] + g_ref[...]
    h = (1.0 - ALPHA) * (dinv_ref[...] * s) + ALPHA * x0_ref[...]
    h_ref[...] = h
    gn_ref[...] = dinv_ref[...] * h


def _combine(sp, g, dinv, x0):
    n, d = g.shape
    bn = _pick_block(n)
    return pl.pallas_call(
        _combine_body,
        grid=(n // bn,),
        in_specs=[
            pl.BlockSpec((2, bn, d), lambda i: (0, i, 0)),
            pl.BlockSpec((bn, d), lambda i: (i, 0)),
            pl.BlockSpec((bn, d), lambda i: (i, 0)),
            pl.BlockSpec((bn, d), lambda i: (i, 0)),
        ],
        out_specs=[
            pl.BlockSpec((bn, d), lambda i: (i, 0)),
            pl.BlockSpec((bn, d), lambda i: (i, 0)),
        ],
        out_shape=[
            jax.ShapeDtypeStruct((n, d), jnp.float32),
            jax.ShapeDtypeStruct((n, d), jnp.float32),
        ],
    )(sp, g, dinv, x0)


def _zero_shared(zbuf, shared, sid, rows_per_tile):
    @pl.loop(0, 128)
    def _(i):
        zbuf[i, :] = jnp.zeros((16,), jnp.float32)
    base = sid * rows_per_tile
    full = rows_per_tile // 128
    rem = rows_per_tile - full * 128

    @pl.loop(0, full)
    def _(j):
        pltpu.sync_copy(zbuf, shared.at[pl.ds(base + j * 128, 128)])
    if rem:
        pltpu.sync_copy(zbuf.at[pl.ds(0, rem)],
                        shared.at[pl.ds(base + full * 128, rem)])


def _writeout(shared, out_hbm, cid, sid, rows_per_tile):
    base = sid * rows_per_tile
    pltpu.sync_copy(shared.at[pl.ds(base, rows_per_tile)],
                    out_hbm.at[cid, pl.ds(base, rows_per_tile)])


def _make_deg_kernel(n_pad, per_w, d):
    rows_per_tile = n_pad // NS
    mesh = plsc.VectorSubcoreMesh(core_axis_name="c", subcore_axis_name="s")

    @functools.partial(
        pl.kernel,
        mesh=mesh,
        out_type=jax.ShapeDtypeStruct((NC, n_pad, d), jnp.float32),
        compiler_params=pltpu.CompilerParams(use_tc_tiling_on_sc=False),
        scratch_types=[
            pltpu.VMEM((per_w, CH), jnp.int32),
            pltpu.VMEM((CH, d), jnp.float32),
            pltpu.VMEM((128, d), jnp.float32),
            pltpu.VMEM_SHARED((n_pad, d), jnp.float32),
        ],
    )
    def deg_kernel(col_hbm, out_hbm, cidx, ones, zbuf, shared):
        cid = lax.axis_index("c")
        sid = lax.axis_index("s")
        w = sid * NC + cid

        _zero_shared(zbuf, shared, sid, rows_per_tile)

        @pl.loop(0, CH)
        def _(i):
            ones[i, :] = jnp.ones((16,), jnp.float32)

        pltpu.sync_copy(col_hbm.at[w], cidx)
        plsc.subcore_barrier()

        @pl.loop(0, per_w)
        def _(j):
            pltpu.sync_copy(ones, shared.at[cidx.at[j]], add=True)

        plsc.subcore_barrier()
        _writeout(shared, out_hbm, cid, sid, rows_per_tile)

    return deg_kernel


def _make_scatter_kernel(n_pad, per_w, d):
    rows_per_tile = n_pad // NS
    mesh = plsc.VectorSubcoreMesh(core_axis_name="c", subcore_axis_name="s")

    @functools.partial(
        pl.kernel,
        mesh=mesh,
        out_type=jax.ShapeDtypeStruct((NC, n_pad, d), jnp.float32),
        compiler_params=pltpu.CompilerParams(use_tc_tiling_on_sc=False),
        scratch_types=[
            pltpu.VMEM((per_w, CH), jnp.int32),
            pltpu.VMEM((per_w, CH), jnp.int32),
            pltpu.VMEM((CH, d), jnp.float32),
            pltpu.VMEM((128, d), jnp.float32),
            pltpu.VMEM_SHARED((n_pad, d), jnp.float32),
            pltpu.SemaphoreType.DMA,
        ],
    )
    def scatter_kernel(g_hbm, row_hbm, col_hbm, out_hbm,
                       ridx, cidx, rows, zbuf, shared, sem):
        cid = lax.axis_index("c")
        sid = lax.axis_index("s")
        w = sid * NC + cid

        _zero_shared(zbuf, shared, sid, rows_per_tile)

        pltpu.sync_copy(row_hbm.at[w], ridx)
        pltpu.sync_copy(col_hbm.at[w], cidx)
        plsc.subcore_barrier()

        @pl.loop(0, per_w)
        def _(j):
            pltpu.async_copy(g_hbm.at[ridx.at[j]], rows, sem).wait()
            pltpu.sync_copy(rows, shared.at[cidx.at[j]], add=True)

        plsc.subcore_barrier()
        _writeout(shared, out_hbm, cid, sid, rows_per_tile)

    return scatter_kernel


def kernel(x, edge_index, W1, b1, W2, b2):
    n = x.shape[0]
    e = edge_index.shape[1]
    d = W2.shape[1]

    row = edge_index[0].astype(jnp.int32)
    col = edge_index[1].astype(jnp.int32)

    per_w = _ceil_div(e, NW * CH)
    e_pad = NW * per_w * CH
    n_pad = _ceil_div(n + 1, NS * 8) * NS * 8

    row_p = jnp.concatenate(
        [row, jnp.zeros((e_pad - e,), jnp.int32)]).reshape(NW, per_w, CH)
    col_p = jnp.concatenate(
        [col, jnp.full((e_pad - e,), n, jnp.int32)]).reshape(NW, per_w, CH)

    deg_k = _make_deg_kernel(n_pad, per_w, d)
    scat_k = _make_scatter_kernel(n_pad, per_w, d)

    h = _mlp(x, W1, b1, W2, b2)
    degp = deg_k(col_p)
    dinv, g = _dinv_g0(degp[:, :n], h)
    x0 = h

    for _ in range(K_STEPS):
        sp = scat_k(g, row_p, col_p)
        h, g = _combine(sp[:, :n], g, dinv, x0)
    return h
